# baseline (speedup 1.0000x reference)
import functools

import jax
import jax.numpy as jnp
from jax import lax
from jax.experimental import pallas as pl
from jax.experimental.pallas import tpu as pltpu
from jax.experimental.pallas import tpu_sc as plsc

N_NODES = 10000
N_EDGES = 320000
D = 128

NC = 2
NS = 16
NW = NC * NS

CHUNK = 128
CPT = 80
E_PAD = NW * CPT * CHUNK
N_PAD = 10240
ROWS_PER_TILE = N_PAD // NS
HALF = CPT // 2


def _sc_body(src_hbm, dst_hbm, x_hbm, part_hbm, acc_sh, sbuf, dbuf,
             rows0, rows1, sem0, sem1):
    c = lax.axis_index("c")
    s = lax.axis_index("s")
    wid = s * NC + c

    def _zero(t, carry):
        i = t // 8
        j = t % 8
        rows0[i, pl.ds(j * 16, 16)] = jnp.zeros((16,), jnp.float32)
        return carry

    lax.fori_loop(0, CHUNK * 8, _zero, None)
    for j in range(ROWS_PER_TILE // CHUNK):
        pltpu.sync_copy(rows0, acc_sh.at[pl.ds(s * ROWS_PER_TILE + j * CHUNK,
                                               CHUNK)])
    plsc.subcore_barrier()

    start = wid * CPT
    for h in range(2):
        pltpu.sync_copy(src_hbm.at[pl.ds(start + h * HALF, HALF)], sbuf)
        pltpu.sync_copy(dst_hbm.at[pl.ds(start + h * HALF, HALF)], dbuf)

        def _step(k, carry):
            pltpu.async_copy(x_hbm.at[sbuf.at[k]], rows0, sem0).wait()
            pltpu.sync_copy(rows0, acc_sh.at[dbuf.at[k]], add=True)
            return carry

        lax.fori_loop(0, HALF, _step, None)
    plsc.subcore_barrier()

    pltpu.sync_copy(acc_sh.at[pl.ds(s * ROWS_PER_TILE, ROWS_PER_TILE)],
                    part_hbm.at[c, pl.ds(s * ROWS_PER_TILE, ROWS_PER_TILE)])


@jax.jit
def _sc_scatter(src2d, dst2d, x):
    mesh = plsc.VectorSubcoreMesh(core_axis_name="c", subcore_axis_name="s")
    return pl.kernel(
        _sc_body,
        mesh=mesh,
        out_type=jax.ShapeDtypeStruct((NC, N_PAD, D), jnp.float32),
        scratch_types=[
            pltpu.VMEM_SHARED((N_PAD, D), jnp.float32),
            pltpu.VMEM((HALF, CHUNK), jnp.int32),
            pltpu.VMEM((HALF, CHUNK), jnp.int32),
            pltpu.VMEM((CHUNK, D), jnp.float32),
            pltpu.VMEM((CHUNK, D), jnp.float32),
            pltpu.SemaphoreType.DMA,
            pltpu.SemaphoreType.DMA,
        ],
    )(src2d, dst2d, x)


def _mm_body(p_ref, w_ref, b_ref, o_ref):
    h = p_ref[0] + p_ref[1]
    o_ref[...] = lax.dot_general(
        h, w_ref[...], (((1,), (1,)), ((), ())),
        preferred_element_type=jnp.float32) + b_ref[...]


def _tc_linear(parts, W, b2d):
    bn = 1000
    grid = N_NODES // bn
    return pl.pallas_call(
        _mm_body,
        grid=(grid,),
        in_specs=[
            pl.BlockSpec((NC, bn, D), lambda i: (0, i, 0)),
            pl.BlockSpec((D, D), lambda i: (0, 0)),
            pl.BlockSpec((1, D), lambda i: (0, 0)),
        ],
        out_specs=pl.BlockSpec((bn, D), lambda i: (i, 0)),
        out_shape=jax.ShapeDtypeStruct((N_NODES, D), jnp.float32),
    )(parts, W, b2d)


def kernel(x, edge_index, W, b):
    src = edge_index[0]
    dst = edge_index[1]
    npad = E_PAD - N_EDGES
    src2d = jnp.concatenate(
        [src, jnp.zeros((npad,), jnp.int32)]).reshape(E_PAD // CHUNK, CHUNK)
    dst2d = jnp.concatenate(
        [dst, jnp.full((npad,), N_NODES, jnp.int32)]).reshape(
            E_PAD // CHUNK, CHUNK)
    parts = _sc_scatter(src2d, dst2d, x)
    return _tc_linear(parts, W, b.reshape(1, D))


# speedup vs baseline: 1.2207x; 1.2207x over previous
import functools

import jax
import jax.numpy as jnp
from jax import lax
from jax.experimental import pallas as pl
from jax.experimental.pallas import tpu as pltpu
from jax.experimental.pallas import tpu_sc as plsc

N_NODES = 10000
N_EDGES = 320000
D = 128

NC = 2
NS = 16
NW = NC * NS

CHUNK = 128
CPT = 80
E_PAD = NW * CPT * CHUNK
N_PAD = 10240
ROWS_PER_TILE = N_PAD // NS


def _sc_body(src_hbm, dst_hbm, x_hbm, part_hbm, acc_sh, srcv0, dstv0,
             srcv1, dstv1, rows0, rows1, sem0, sem1):
    c = lax.axis_index("c")
    s = lax.axis_index("s")
    wid = s * NC + c

    def _zero(t, carry):
        i = t // 8
        j = t % 8
        rows0[i, pl.ds(j * 16, 16)] = jnp.zeros((16,), jnp.float32)
        return carry

    lax.fori_loop(0, CHUNK * 8, _zero, None)
    for j in range(ROWS_PER_TILE // CHUNK):
        pltpu.sync_copy(rows0, acc_sh.at[pl.ds(s * ROWS_PER_TILE + j * CHUNK,
                                               CHUNK)])
    plsc.subcore_barrier()

    def _pair(g, carry):
        offa = (wid + (2 * g) * NW) * CHUNK
        offb = offa + NW * CHUNK
        pltpu.sync_copy(src_hbm.at[pl.ds(offa, CHUNK)], srcv0)
        pltpu.sync_copy(dst_hbm.at[pl.ds(offa, CHUNK)], dstv0)
        cpa = pltpu.async_copy(x_hbm.at[srcv0], rows0, sem0)
        pltpu.sync_copy(src_hbm.at[pl.ds(offb, CHUNK)], srcv1)
        pltpu.sync_copy(dst_hbm.at[pl.ds(offb, CHUNK)], dstv1)
        cpb = pltpu.async_copy(x_hbm.at[srcv1], rows1, sem1)
        cpa.wait()
        pltpu.sync_copy(rows0, acc_sh.at[dstv0], add=True)
        cpb.wait()
        pltpu.sync_copy(rows1, acc_sh.at[dstv1], add=True)
        return carry

    lax.fori_loop(0, CPT // 2, _pair, None)
    plsc.subcore_barrier()

    pltpu.sync_copy(acc_sh.at[pl.ds(s * ROWS_PER_TILE, ROWS_PER_TILE)],
                    part_hbm.at[c, pl.ds(s * ROWS_PER_TILE, ROWS_PER_TILE)])


@jax.jit
def _sc_scatter(src2d, dst2d, x):
    mesh = plsc.VectorSubcoreMesh(core_axis_name="c", subcore_axis_name="s")
    return pl.kernel(
        _sc_body,
        mesh=mesh,
        out_type=jax.ShapeDtypeStruct((NC, N_PAD, D), jnp.float32),
        scratch_types=[
            pltpu.VMEM_SHARED((N_PAD, D), jnp.float32),
            pltpu.VMEM((CHUNK,), jnp.int32),
            pltpu.VMEM((CHUNK,), jnp.int32),
            pltpu.VMEM((CHUNK,), jnp.int32),
            pltpu.VMEM((CHUNK,), jnp.int32),
            pltpu.VMEM((CHUNK, D), jnp.float32),
            pltpu.VMEM((CHUNK, D), jnp.float32),
            pltpu.SemaphoreType.DMA,
            pltpu.SemaphoreType.DMA,
        ],
    )(src2d, dst2d, x)


def _mm_body(p_ref, w_ref, b_ref, o_ref):
    h = p_ref[0] + p_ref[1]
    o_ref[...] = lax.dot_general(
        h, w_ref[...], (((1,), (1,)), ((), ())),
        preferred_element_type=jnp.float32) + b_ref[...]


def _tc_linear(parts, W, b2d):
    bn = 1000
    grid = N_NODES // bn
    return pl.pallas_call(
        _mm_body,
        grid=(grid,),
        in_specs=[
            pl.BlockSpec((NC, bn, D), lambda i: (0, i, 0)),
            pl.BlockSpec((D, D), lambda i: (0, 0)),
            pl.BlockSpec((1, D), lambda i: (0, 0)),
        ],
        out_specs=pl.BlockSpec((bn, D), lambda i: (i, 0)),
        out_shape=jax.ShapeDtypeStruct((N_NODES, D), jnp.float32),
    )(parts, W, b2d)


def kernel(x, edge_index, W, b):
    src = edge_index[0]
    dst = edge_index[1]
    npad = E_PAD - N_EDGES
    src_p = jnp.concatenate([src, jnp.zeros((npad,), jnp.int32)])
    dst_p = jnp.concatenate([dst, jnp.full((npad,), N_NODES, jnp.int32)])
    parts = _sc_scatter(src_p, dst_p, x)
    return _tc_linear(parts, W, b.reshape(1, D))


# speedup vs baseline: 1.2216x; 1.0007x over previous
import functools

import jax
import jax.numpy as jnp
from jax import lax
from jax.experimental import pallas as pl
from jax.experimental.pallas import tpu as pltpu
from jax.experimental.pallas import tpu_sc as plsc

N_NODES = 10000
N_EDGES = 320000
D = 128

NC = 2
NS = 16
NW = NC * NS

CHUNK = 128
CPT = 80
E_PAD = NW * CPT * CHUNK
N_PAD = 10240
ROWS_PER_TILE = N_PAD // NS


def _sc_body(src_hbm, dst_hbm, x_hbm, part_hbm, acc_sh, srcv0, dstv0,
             srcv1, dstv1, rows0, rows1, sem0, sem1):
    c = lax.axis_index("c")
    s = lax.axis_index("s")
    wid = s * NC + c

    def _zero(t, carry):
        i = t // 8
        j = t % 8
        rows0[i, pl.ds(j * 16, 16)] = jnp.zeros((16,), jnp.float32)
        return carry

    lax.fori_loop(0, CHUNK * 8, _zero, None)
    for j in range(ROWS_PER_TILE // CHUNK):
        pltpu.sync_copy(rows0, acc_sh.at[pl.ds(s * ROWS_PER_TILE + j * CHUNK,
                                               CHUNK)])
    plsc.subcore_barrier()

    def _pair(g, carry):
        offa = (wid + (2 * g) * NW) * CHUNK
        offb = offa + NW * CHUNK
        pltpu.sync_copy(src_hbm.at[pl.ds(offa, CHUNK)], srcv0)
        pltpu.sync_copy(dst_hbm.at[pl.ds(offa, CHUNK)], dstv0)
        cpa = pltpu.async_copy(x_hbm.at[srcv0], rows0, sem0)
        pltpu.sync_copy(src_hbm.at[pl.ds(offb, CHUNK)], srcv1)
        pltpu.sync_copy(dst_hbm.at[pl.ds(offb, CHUNK)], dstv1)
        cpb = pltpu.async_copy(x_hbm.at[srcv1], rows1, sem1)
        cpa.wait()
        pltpu.sync_copy(rows0, acc_sh.at[dstv0], add=True)
        cpb.wait()
        pltpu.sync_copy(rows1, acc_sh.at[dstv1], add=True)
        return carry

    lax.fori_loop(0, CPT // 2, _pair, None)
    plsc.subcore_barrier()

    pltpu.sync_copy(acc_sh.at[pl.ds(s * ROWS_PER_TILE, ROWS_PER_TILE)],
                    part_hbm.at[c, pl.ds(s * ROWS_PER_TILE, ROWS_PER_TILE)])


@jax.jit
def _sc_scatter(src2d, dst2d, x):
    mesh = plsc.VectorSubcoreMesh(core_axis_name="c", subcore_axis_name="s")
    return pl.kernel(
        _sc_body,
        mesh=mesh,
        out_type=jax.ShapeDtypeStruct((NC, N_PAD, D), jnp.float32),
        scratch_types=[
            pltpu.VMEM_SHARED((N_PAD, D), jnp.float32),
            pltpu.VMEM((CHUNK,), jnp.int32),
            pltpu.VMEM((CHUNK,), jnp.int32),
            pltpu.VMEM((CHUNK,), jnp.int32),
            pltpu.VMEM((CHUNK,), jnp.int32),
            pltpu.VMEM((CHUNK, D), jnp.float32),
            pltpu.VMEM((CHUNK, D), jnp.float32),
            pltpu.SemaphoreType.DMA,
            pltpu.SemaphoreType.DMA,
        ],
    )(src2d, dst2d, x)


def _mm_body(p_ref, w_ref, b_ref, o_ref):
    h = p_ref[0] + p_ref[1]
    o_ref[...] = lax.dot_general(
        h, w_ref[...], (((1,), (1,)), ((), ())),
        preferred_element_type=jnp.float32) + b_ref[...]


def _tc_linear(parts, W, b2d):
    bn = 1000
    grid = N_NODES // bn
    return pl.pallas_call(
        _mm_body,
        grid=(grid,),
        in_specs=[
            pl.BlockSpec((NC, bn, D), lambda i: (0, i, 0)),
            pl.BlockSpec((D, D), lambda i: (0, 0)),
            pl.BlockSpec((1, D), lambda i: (0, 0)),
        ],
        out_specs=pl.BlockSpec((bn, D), lambda i: (i, 0)),
        out_shape=jax.ShapeDtypeStruct((N_NODES, D), jnp.float32),
    )(parts, W, b2d)


def kernel(x, edge_index, W, b):
    src = edge_index[0]
    dst = edge_index[1]
    npad = E_PAD - N_EDGES
    src_p = jnp.concatenate([src, jnp.zeros((npad,), jnp.int32)])
    dst_p = jnp.concatenate(
        [dst, N_NODES + jnp.arange(npad, dtype=jnp.int32) % (N_PAD - N_NODES)])
    parts = _sc_scatter(src_p, dst_p, x)
    return _tc_linear(parts, W, b.reshape(1, D))


# speedup vs baseline: 2.7975x; 2.2901x over previous
import functools

import jax
import jax.numpy as jnp
from jax import lax
from jax.experimental import pallas as pl
from jax.experimental.pallas import tpu as pltpu
from jax.experimental.pallas import tpu_sc as plsc

N_NODES = 10000
N_EDGES = 320000
D = 128

NC = 2
NS = 16
NW = NC * NS

CHUNK = 128
NCHUNKS = N_EDGES // CHUNK
N_PAD = 10240
ROWS_PER_TILE = N_PAD // NS


def _sc_body(src_hbm, dst_hbm, x_hbm, part_hbm, acc_sh, srcv0, dstv0,
             srcv1, dstv1, rows0, rows1, sem0, sem1):
    c = lax.axis_index("c")
    s = lax.axis_index("s")
    wid = s * NC + c

    def _zero(t, carry):
        i = t // 8
        j = t % 8
        rows0[i, pl.ds(j * 16, 16)] = jnp.zeros((16,), jnp.float32)
        return carry

    lax.fori_loop(0, CHUNK * 8, _zero, None)
    for j in range(ROWS_PER_TILE // CHUNK):
        pltpu.sync_copy(rows0, acc_sh.at[pl.ds(s * ROWS_PER_TILE + j * CHUNK,
                                               CHUNK)])
    plsc.subcore_barrier()

    def _off(k):
        return jnp.minimum(wid + k * NW, NCHUNKS - 1) * CHUNK

    pltpu.sync_copy(src_hbm.at[pl.ds(_off(0), CHUNK)], srcv0)
    pltpu.sync_copy(dst_hbm.at[pl.ds(_off(0), CHUNK)], dstv0)

    def _pair(g, carry):
        ka = 2 * g
        pltpu.async_copy(src_hbm.at[pl.ds(_off(ka + 1), CHUNK)], srcv1, sem1)
        pltpu.async_copy(dst_hbm.at[pl.ds(_off(ka + 1), CHUNK)], dstv1, sem1)
        pltpu.async_copy(x_hbm.at[srcv0], rows0, sem0).wait()
        pltpu.sync_copy(rows0, acc_sh.at[dstv0], add=True)
        pltpu.make_async_copy(src_hbm.at[pl.ds(0, CHUNK)], srcv1, sem1).wait()
        pltpu.make_async_copy(dst_hbm.at[pl.ds(0, CHUNK)], dstv1, sem1).wait()
        pltpu.async_copy(src_hbm.at[pl.ds(_off(ka + 2), CHUNK)], srcv0, sem1)
        pltpu.async_copy(dst_hbm.at[pl.ds(_off(ka + 2), CHUNK)], dstv0, sem1)
        pltpu.async_copy(x_hbm.at[srcv1], rows1, sem0).wait()
        pltpu.sync_copy(rows1, acc_sh.at[dstv1], add=True)
        pltpu.make_async_copy(src_hbm.at[pl.ds(0, CHUNK)], srcv0, sem1).wait()
        pltpu.make_async_copy(dst_hbm.at[pl.ds(0, CHUNK)], dstv0, sem1).wait()
        return carry

    lax.fori_loop(0, NCHUNKS // NW // 2, _pair, None)
    @pl.when(wid < NCHUNKS % NW)
    def _tail():
        pltpu.async_copy(x_hbm.at[srcv0], rows0, sem0).wait()
        pltpu.sync_copy(rows0, acc_sh.at[dstv0], add=True)

    plsc.subcore_barrier()

    pltpu.sync_copy(acc_sh.at[pl.ds(s * ROWS_PER_TILE, ROWS_PER_TILE)],
                    part_hbm.at[c, pl.ds(s * ROWS_PER_TILE, ROWS_PER_TILE)])


@jax.jit
def _sc_scatter(src2d, dst2d, x):
    mesh = plsc.VectorSubcoreMesh(core_axis_name="c", subcore_axis_name="s")
    return pl.kernel(
        _sc_body,
        mesh=mesh,
        out_type=jax.ShapeDtypeStruct((NC, N_PAD, D), jnp.float32),
        scratch_types=[
            pltpu.VMEM_SHARED((N_PAD, D), jnp.float32),
            pltpu.VMEM((CHUNK,), jnp.int32),
            pltpu.VMEM((CHUNK,), jnp.int32),
            pltpu.VMEM((CHUNK,), jnp.int32),
            pltpu.VMEM((CHUNK,), jnp.int32),
            pltpu.VMEM((CHUNK, D), jnp.float32),
            pltpu.VMEM((CHUNK, D), jnp.float32),
            pltpu.SemaphoreType.DMA,
            pltpu.SemaphoreType.DMA,
        ],
    )(src2d, dst2d, x)


def _mm_body(p_ref, w_ref, b_ref, o_ref):
    h = p_ref[0] + p_ref[1]
    o_ref[...] = lax.dot_general(
        h, w_ref[...], (((1,), (1,)), ((), ())),
        preferred_element_type=jnp.float32) + b_ref[...]


def _tc_linear(parts, W, b2d):
    bn = 1000
    grid = N_NODES // bn
    return pl.pallas_call(
        _mm_body,
        grid=(grid,),
        in_specs=[
            pl.BlockSpec((NC, bn, D), lambda i: (0, i, 0)),
            pl.BlockSpec((D, D), lambda i: (0, 0)),
            pl.BlockSpec((1, D), lambda i: (0, 0)),
        ],
        out_specs=pl.BlockSpec((bn, D), lambda i: (i, 0)),
        out_shape=jax.ShapeDtypeStruct((N_NODES, D), jnp.float32),
    )(parts, W, b2d)


def kernel(x, edge_index, W, b):
    src = edge_index[0]
    dst = edge_index[1]
    parts = _sc_scatter(src, dst, x)
    return _tc_linear(parts, W, b.reshape(1, D))


# speedup vs baseline: 3.7477x; 1.3397x over previous
import functools

import jax
import jax.numpy as jnp
from jax import lax
from jax.experimental import pallas as pl
from jax.experimental.pallas import tpu as pltpu
from jax.experimental.pallas import tpu_sc as plsc

N_NODES = 10000
N_EDGES = 320000
D = 128

NC = 2
NS = 16
NW = NC * NS

CHUNK = 128
NCHUNKS = N_EDGES // CHUNK
N_PAD = 10240
ROWS_PER_TILE = N_PAD // NS


def _sc_body(ei_hbm, x_hbm, part_hbm, acc_sh, srcv0, dstv0,
             srcv1, dstv1, dsts0, dsts1, rows0, rows1, semi, semg, sems):
    c = lax.axis_index("c")
    s = lax.axis_index("s")
    wid = s * NC + c

    def _zero(t, carry):
        i = t // 8
        j = t % 8
        rows0[i, pl.ds(j * 16, 16)] = jnp.zeros((16,), jnp.float32)
        return carry

    lax.fori_loop(0, CHUNK * 8, _zero, None)
    for j in range(ROWS_PER_TILE // CHUNK):
        pltpu.sync_copy(rows0, acc_sh.at[pl.ds(s * ROWS_PER_TILE + j * CHUNK,
                                               CHUNK)])
    plsc.subcore_barrier()

    def _off(k):
        return jnp.minimum(wid + k * NW, NCHUNKS - 1) * CHUNK

    def _snap(dv, ds_):
        for j in range(CHUNK // 16):
            ds_[pl.ds(j * 16, 16)] = dv[pl.ds(j * 16, 16)]

    pltpu.sync_copy(ei_hbm.at[pl.ds(_off(0), CHUNK)], srcv0)
    pltpu.sync_copy(ei_hbm.at[pl.ds(N_EDGES + _off(0), CHUNK)], dstv0)

    def _pair(g, carry):
        ka = 2 * g
        pltpu.async_copy(ei_hbm.at[pl.ds(_off(ka + 1), CHUNK)], srcv1, semi)
        pltpu.async_copy(ei_hbm.at[pl.ds(N_EDGES + _off(ka + 1), CHUNK)],
                         dstv1, semi)
        pltpu.async_copy(x_hbm.at[srcv0], rows0, semg).wait()

        @pl.when(g > 0)
        def _wait_prev():
            pltpu.make_async_copy(x_hbm.at[pl.ds(0, CHUNK)], rows1,
                                  sems).wait()

        _snap(dstv0, dsts0)
        pltpu.async_copy(rows0, acc_sh.at[dsts0], add=True, sem=sems)
        pltpu.make_async_copy(ei_hbm.at[pl.ds(0, CHUNK)], srcv1, semi).wait()
        pltpu.make_async_copy(ei_hbm.at[pl.ds(0, CHUNK)], dstv1, semi).wait()
        pltpu.async_copy(ei_hbm.at[pl.ds(_off(ka + 2), CHUNK)], srcv0, semi)
        pltpu.async_copy(ei_hbm.at[pl.ds(N_EDGES + _off(ka + 2), CHUNK)],
                         dstv0, semi)
        pltpu.async_copy(x_hbm.at[srcv1], rows1, semg).wait()
        pltpu.make_async_copy(x_hbm.at[pl.ds(0, CHUNK)], rows0, sems).wait()
        _snap(dstv1, dsts1)
        pltpu.async_copy(rows1, acc_sh.at[dsts1], add=True, sem=sems)
        pltpu.make_async_copy(ei_hbm.at[pl.ds(0, CHUNK)], srcv0, semi).wait()
        pltpu.make_async_copy(ei_hbm.at[pl.ds(0, CHUNK)], dstv0, semi).wait()
        return carry

    lax.fori_loop(0, NCHUNKS // NW // 2, _pair, None)
    pltpu.make_async_copy(x_hbm.at[pl.ds(0, CHUNK)], rows1, sems).wait()
    @pl.when(wid < NCHUNKS % NW)
    def _tail():
        pltpu.async_copy(x_hbm.at[srcv0], rows0, semg).wait()
        pltpu.sync_copy(rows0, acc_sh.at[dstv0], add=True)

    plsc.subcore_barrier()

    pltpu.sync_copy(acc_sh.at[pl.ds(s * ROWS_PER_TILE, ROWS_PER_TILE)],
                    part_hbm.at[c, pl.ds(s * ROWS_PER_TILE, ROWS_PER_TILE)])


@jax.jit
def _sc_scatter(ei_flat, x):
    mesh = plsc.VectorSubcoreMesh(core_axis_name="c", subcore_axis_name="s")
    return pl.kernel(
        _sc_body,
        mesh=mesh,
        out_type=jax.ShapeDtypeStruct((NC, N_PAD, D), jnp.float32),
        scratch_types=[
            pltpu.VMEM_SHARED((N_PAD, D), jnp.float32),
            pltpu.VMEM((CHUNK,), jnp.int32),
            pltpu.VMEM((CHUNK,), jnp.int32),
            pltpu.VMEM((CHUNK,), jnp.int32),
            pltpu.VMEM((CHUNK,), jnp.int32),
            pltpu.VMEM((CHUNK,), jnp.int32),
            pltpu.VMEM((CHUNK,), jnp.int32),
            pltpu.VMEM((CHUNK, D), jnp.float32),
            pltpu.VMEM((CHUNK, D), jnp.float32),
            pltpu.SemaphoreType.DMA,
            pltpu.SemaphoreType.DMA,
            pltpu.SemaphoreType.DMA,
        ],
    )(ei_flat, x)


def _mm_body(p_ref, w_ref, b_ref, o_ref):
    h = p_ref[0] + p_ref[1]
    o_ref[...] = lax.dot_general(
        h, w_ref[...], (((1,), (1,)), ((), ())),
        preferred_element_type=jnp.float32) + b_ref[...]


def _tc_linear(parts, W, b2d):
    bn = 1000
    grid = N_NODES // bn
    return pl.pallas_call(
        _mm_body,
        grid=(grid,),
        in_specs=[
            pl.BlockSpec((NC, bn, D), lambda i: (0, i, 0)),
            pl.BlockSpec((D, D), lambda i: (0, 0)),
            pl.BlockSpec((1, D), lambda i: (0, 0)),
        ],
        out_specs=pl.BlockSpec((bn, D), lambda i: (i, 0)),
        out_shape=jax.ShapeDtypeStruct((N_NODES, D), jnp.float32),
    )(parts, W, b2d)


def kernel(x, edge_index, W, b):
    parts = _sc_scatter(edge_index.reshape(-1), x)
    return _tc_linear(parts, W, b.reshape(1, D))
